# baseline (device time: 45111 ns/iter reference)
import jax
import jax.numpy as jnp
from jax import lax
from jax.experimental import pallas as pl
from jax.experimental.pallas import tpu as pltpu

M = 1024
N = 1024
D = 4096
H = M // 2
RB = 2
WB = 8
CR = H // RB
CW = N // WB
KC = RB * WB


def kernel(dy, W):
    def body(dy_ref, w_ref, out_ref, wvmem, dybuf, pbuf, ybuf,
             w_sems, dy_sems, out_sems,
             ysend_sems, yrecv_sems, xsend_sems, xrecv_sems):
        my_x = lax.axis_index("x")
        my_y = lax.axis_index("y")

        barrier_sem = pltpu.get_barrier_semaphore()
        pl.semaphore_signal(
            barrier_sem, inc=1,
            device_id=(my_x, 1 - my_y), device_id_type=pl.DeviceIdType.MESH)
        pl.semaphore_signal(
            barrier_sem, inc=1,
            device_id=(1 - my_x, my_y), device_id_type=pl.DeviceIdType.MESH)

        row0 = my_x * H

        def dy_load(r):
            return pltpu.make_async_copy(
                dy_ref.at[pl.ds(row0 + r * CR, CR)], dybuf.at[r],
                dy_sems.at[r])

        def w_load(b):
            return pltpu.make_async_copy(
                w_ref.at[pl.ds(b * CW, CW)], wvmem.at[pl.ds(b * CW, CW)],
                w_sems.at[b])

        dy_load(0).start()
        w_load(0).start()
        dy_load(1).start()
        for b in range(1, WB):
            w_load(b).start()

        def y_copy(k):
            return pltpu.make_async_remote_copy(
                src_ref=pbuf.at[k],
                dst_ref=ybuf.at[k],
                send_sem=ysend_sems.at[k],
                recv_sem=yrecv_sems.at[k],
                device_id=(my_x, 1 - my_y),
                device_id_type=pl.DeviceIdType.MESH,
            )

        def tile_rows(r):
            return pl.ds(row0 + r * CR, CR)

        def x_copy(k):
            b, r = divmod(k, RB)
            return pltpu.make_async_remote_copy(
                src_ref=pbuf.at[k],
                dst_ref=out_ref.at[tile_rows(r), pl.ds(b * CW, CW)],
                send_sem=xsend_sems.at[k],
                recv_sem=xrecv_sems.at[k],
                device_id=(1 - my_x, my_y),
                device_id_type=pl.DeviceIdType.MESH,
            )

        def out_copy(k):
            b, r = divmod(k, RB)
            return pltpu.make_async_copy(
                pbuf.at[k],
                out_ref.at[tile_rows(r), pl.ds(b * CW, CW)],
                out_sems.at[k],
            )

        for b in range(WB):
            w_load(b).wait()
            for r in range(RB):
                if b == 0:
                    dy_load(r).wait()
                k = b * RB + r
                p = lax.dot_general(
                    dybuf[r], wvmem[pl.ds(b * CW, CW), :],
                    dimension_numbers=(((1,), (1,)), ((), ())),
                    preferred_element_type=jnp.float32,
                )
                pbuf[k, :, :] = p.astype(jnp.bfloat16)
                if k == 0:
                    pl.semaphore_wait(barrier_sem, 2)
                y_copy(k).start()

        for k in range(KC):
            yc = y_copy(k)
            yc.wait_send()
            yc.wait_recv()
            pbuf[k, :, :] = pbuf[k, :, :] + ybuf[k, :, :]
            out_copy(k).start()
            x_copy(k).start()

        for k in range(KC):
            x_copy(k).wait()
            out_copy(k).wait()

    return pl.pallas_call(
        body,
        out_shape=jax.ShapeDtypeStruct((M, N), jnp.bfloat16),
        in_specs=[
            pl.BlockSpec(memory_space=pl.ANY),
            pl.BlockSpec(memory_space=pl.ANY),
        ],
        out_specs=pl.BlockSpec(memory_space=pl.ANY),
        scratch_shapes=[
            pltpu.VMEM((N, D), jnp.float32),
            pltpu.VMEM((RB, CR, D), jnp.float32),
            pltpu.VMEM((KC, CR, CW), jnp.bfloat16),
            pltpu.VMEM((KC, CR, CW), jnp.bfloat16),
            pltpu.SemaphoreType.DMA((WB,)),
            pltpu.SemaphoreType.DMA((RB,)),
            pltpu.SemaphoreType.DMA((KC,)),
            pltpu.SemaphoreType.DMA((KC,)),
            pltpu.SemaphoreType.DMA((KC,)),
            pltpu.SemaphoreType.DMA((KC,)),
            pltpu.SemaphoreType.DMA((KC,)),
        ],
        compiler_params=pltpu.CompilerParams(collective_id=0),
    )(dy, W)


# device time: 36099 ns/iter; 1.2496x vs baseline; 1.2496x over previous
import jax
import jax.numpy as jnp
from jax import lax
from jax.experimental import pallas as pl
from jax.experimental.pallas import tpu as pltpu

M = 1024
N = 1024
D = 4096
H = M // 2
RB = 2
WB = 4
CR = H // RB
CW = N // WB
KC = RB * WB


def kernel(dy, W):
    def body(dy_ref, w_ref, out_ref, wvmem, dybuf, pbuf, ybuf,
             w_sems, dy_sems, out_sems,
             ysend_sems, yrecv_sems, xsend_sems, xrecv_sems):
        my_x = lax.axis_index("x")
        my_y = lax.axis_index("y")

        barrier_sem = pltpu.get_barrier_semaphore()
        pl.semaphore_signal(
            barrier_sem, inc=1,
            device_id=(my_x, 1 - my_y), device_id_type=pl.DeviceIdType.MESH)
        pl.semaphore_signal(
            barrier_sem, inc=1,
            device_id=(1 - my_x, my_y), device_id_type=pl.DeviceIdType.MESH)

        row0 = my_x * H

        def dy_load(r):
            return pltpu.make_async_copy(
                dy_ref.at[pl.ds(row0 + r * CR, CR)], dybuf.at[r],
                dy_sems.at[r])

        def w_load(b):
            return pltpu.make_async_copy(
                w_ref.at[pl.ds(b * CW, CW)], wvmem.at[pl.ds(b * CW, CW)],
                w_sems.at[b])

        dy_load(0).start()
        w_load(0).start()
        dy_load(1).start()
        for b in range(1, WB):
            w_load(b).start()

        def y_copy(k):
            return pltpu.make_async_remote_copy(
                src_ref=pbuf.at[k],
                dst_ref=ybuf.at[k],
                send_sem=ysend_sems.at[k],
                recv_sem=yrecv_sems.at[k],
                device_id=(my_x, 1 - my_y),
                device_id_type=pl.DeviceIdType.MESH,
            )

        def tile_rows(r):
            return pl.ds(row0 + r * CR, CR)

        def x_copy(k):
            b, r = divmod(k, RB)
            return pltpu.make_async_remote_copy(
                src_ref=pbuf.at[k],
                dst_ref=out_ref.at[tile_rows(r), pl.ds(b * CW, CW)],
                send_sem=xsend_sems.at[k],
                recv_sem=xrecv_sems.at[k],
                device_id=(1 - my_x, my_y),
                device_id_type=pl.DeviceIdType.MESH,
            )

        def out_copy(k):
            b, r = divmod(k, RB)
            return pltpu.make_async_copy(
                pbuf.at[k],
                out_ref.at[tile_rows(r), pl.ds(b * CW, CW)],
                out_sems.at[k],
            )

        for b in range(WB):
            w_load(b).wait()
            for r in range(RB):
                if b == 0:
                    dy_load(r).wait()
                k = b * RB + r
                p = lax.dot_general(
                    dybuf[r], wvmem[pl.ds(b * CW, CW), :],
                    dimension_numbers=(((1,), (1,)), ((), ())),
                    preferred_element_type=jnp.float32,
                )
                pbuf[k, :, :] = p.astype(jnp.bfloat16)
                if k == 0:
                    pl.semaphore_wait(barrier_sem, 2)
                y_copy(k).start()

        for k in range(KC):
            yc = y_copy(k)
            yc.wait_send()
            yc.wait_recv()
            pbuf[k, :, :] = pbuf[k, :, :] + ybuf[k, :, :]
            out_copy(k).start()
            x_copy(k).start()

        for k in range(KC):
            x_copy(k).wait()
            out_copy(k).wait()

    return pl.pallas_call(
        body,
        out_shape=jax.ShapeDtypeStruct((M, N), jnp.bfloat16),
        in_specs=[
            pl.BlockSpec(memory_space=pl.ANY),
            pl.BlockSpec(memory_space=pl.ANY),
        ],
        out_specs=pl.BlockSpec(memory_space=pl.ANY),
        scratch_shapes=[
            pltpu.VMEM((N, D), jnp.float32),
            pltpu.VMEM((RB, CR, D), jnp.float32),
            pltpu.VMEM((KC, CR, CW), jnp.bfloat16),
            pltpu.VMEM((KC, CR, CW), jnp.bfloat16),
            pltpu.SemaphoreType.DMA((WB,)),
            pltpu.SemaphoreType.DMA((RB,)),
            pltpu.SemaphoreType.DMA((KC,)),
            pltpu.SemaphoreType.DMA((KC,)),
            pltpu.SemaphoreType.DMA((KC,)),
            pltpu.SemaphoreType.DMA((KC,)),
            pltpu.SemaphoreType.DMA((KC,)),
        ],
        compiler_params=pltpu.CompilerParams(collective_id=0),
    )(dy, W)


# device time: 35836 ns/iter; 1.2588x vs baseline; 1.0073x over previous
import jax
import jax.numpy as jnp
from jax import lax
from jax.experimental import pallas as pl
from jax.experimental.pallas import tpu as pltpu

M = 1024
N = 1024
D = 4096
H = M // 2
RB = 2
WB = 4
CR = H // RB
CW = N // WB
KC = RB * WB


def kernel(dy, W):
    def body(dy_ref, w_ref, out_ref, *refs):
        wvmem = refs[0]
        dybuf = refs[1]
        pbufs = refs[2:2 + KC]
        rbufs = refs[2 + KC:2 + 2 * KC]
        ybuf = refs[2 + 2 * KC]
        (w_sems, dy_sems, out_sems,
         ysend_sems, yrecv_sems, xsend_sems, xrecv_sems) = refs[3 + 2 * KC:]

        my_x = lax.axis_index("x")
        my_y = lax.axis_index("y")

        barrier_sem = pltpu.get_barrier_semaphore()
        pl.semaphore_signal(
            barrier_sem, inc=1,
            device_id=(my_x, 1 - my_y), device_id_type=pl.DeviceIdType.MESH)
        pl.semaphore_signal(
            barrier_sem, inc=1,
            device_id=(1 - my_x, my_y), device_id_type=pl.DeviceIdType.MESH)

        row0 = my_x * H

        def dy_load(r):
            return pltpu.make_async_copy(
                dy_ref.at[pl.ds(row0 + r * CR, CR)], dybuf.at[r],
                dy_sems.at[r])

        def w_load(b):
            return pltpu.make_async_copy(
                w_ref.at[pl.ds(b * CW, CW)], wvmem.at[pl.ds(b * CW, CW)],
                w_sems.at[b])

        dy_load(0).start()
        w_load(0).start()
        dy_load(1).start()
        for b in range(1, WB):
            w_load(b).start()

        def y_copy(k):
            return pltpu.make_async_remote_copy(
                src_ref=pbufs[k],
                dst_ref=ybuf.at[k],
                send_sem=ysend_sems.at[k],
                recv_sem=yrecv_sems.at[k],
                device_id=(my_x, 1 - my_y),
                device_id_type=pl.DeviceIdType.MESH,
            )

        def out_slices(k):
            b, r = divmod(k, RB)
            return (pl.ds(row0 + r * CR, CR), pl.ds(b * CW, CW))

        def x_copy(k):
            return pltpu.make_async_remote_copy(
                src_ref=rbufs[k],
                dst_ref=out_ref.at[out_slices(k)],
                send_sem=xsend_sems.at[k],
                recv_sem=xrecv_sems.at[k],
                device_id=(1 - my_x, my_y),
                device_id_type=pl.DeviceIdType.MESH,
            )

        def out_copy(k):
            return pltpu.make_async_copy(
                rbufs[k], out_ref.at[out_slices(k)], out_sems.at[k])

        for b in range(WB):
            w_load(b).wait()
            for r in range(RB):
                if b == 0:
                    dy_load(r).wait()
                k = b * RB + r
                p = lax.dot_general(
                    dybuf[r], wvmem[pl.ds(b * CW, CW), :],
                    dimension_numbers=(((1,), (1,)), ((), ())),
                    preferred_element_type=jnp.float32,
                )
                pbufs[k][...] = p.astype(jnp.bfloat16)
                if k == 0:
                    pl.semaphore_wait(barrier_sem, 2)
                y_copy(k).start()

        for k in range(KC):
            y_copy(k).wait_recv()
            rbufs[k][...] = pbufs[k][...] + ybuf[k]
            out_copy(k).start()
            x_copy(k).start()

        for k in range(KC):
            y_copy(k).wait_send()
            x_copy(k).wait()
            out_copy(k).wait()

    return pl.pallas_call(
        body,
        out_shape=jax.ShapeDtypeStruct((M, N), jnp.bfloat16),
        in_specs=[
            pl.BlockSpec(memory_space=pl.ANY),
            pl.BlockSpec(memory_space=pl.ANY),
        ],
        out_specs=pl.BlockSpec(memory_space=pl.ANY),
        scratch_shapes=(
            [
                pltpu.VMEM((N, D), jnp.float32),
                pltpu.VMEM((RB, CR, D), jnp.float32),
            ]
            + [pltpu.VMEM((CR, CW), jnp.bfloat16)] * KC
            + [pltpu.VMEM((CR, CW), jnp.bfloat16)] * KC
            + [
                pltpu.VMEM((KC, CR, CW), jnp.bfloat16),
                pltpu.SemaphoreType.DMA((WB,)),
                pltpu.SemaphoreType.DMA((RB,)),
                pltpu.SemaphoreType.DMA((KC,)),
                pltpu.SemaphoreType.DMA((KC,)),
                pltpu.SemaphoreType.DMA((KC,)),
                pltpu.SemaphoreType.DMA((KC,)),
                pltpu.SemaphoreType.DMA((KC,)),
            ]
        ),
        compiler_params=pltpu.CompilerParams(collective_id=0),
    )(dy, W)


# device time: 33833 ns/iter; 1.3333x vs baseline; 1.0592x over previous
import jax
import jax.numpy as jnp
from jax import lax
from jax.experimental import pallas as pl
from jax.experimental.pallas import tpu as pltpu

M = 1024
N = 1024
H = M // 2
K_CMP = 2
K_COM = 16
CC = H // K_CMP
R = H // K_COM


def kernel(dy, W):
    def body(dy_ref, w_ref, out_ref, pbuf, ybuf,
             ysend_sems, yrecv_sems, xsend_sems, xrecv_sems):
        my_x = lax.axis_index("x")
        my_y = lax.axis_index("y")

        barrier_sem = pltpu.get_barrier_semaphore()
        pl.semaphore_signal(
            barrier_sem, inc=1,
            device_id=(my_x, 1 - my_y), device_id_type=pl.DeviceIdType.MESH)
        pl.semaphore_signal(
            barrier_sem, inc=1,
            device_id=(1 - my_x, my_y), device_id_type=pl.DeviceIdType.MESH)

        row0 = my_x * H

        def y_copy(k):
            return pltpu.make_async_remote_copy(
                src_ref=pbuf.at[pl.ds(k * R, R)],
                dst_ref=ybuf.at[pl.ds(k * R, R)],
                send_sem=ysend_sems.at[k],
                recv_sem=yrecv_sems.at[k],
                device_id=(my_x, 1 - my_y),
                device_id_type=pl.DeviceIdType.MESH,
            )

        def x_copy(k):
            return pltpu.make_async_remote_copy(
                src_ref=out_ref.at[pl.ds(row0 + k * R, R)],
                dst_ref=out_ref.at[pl.ds(row0 + k * R, R)],
                send_sem=xsend_sems.at[k],
                recv_sem=xrecv_sems.at[k],
                device_id=(1 - my_x, my_y),
                device_id_type=pl.DeviceIdType.MESH,
            )

        for c in range(K_CMP):
            a = dy_ref[pl.ds(row0 + c * CC, CC), :]
            p = lax.dot_general(
                a, w_ref[...],
                dimension_numbers=(((1,), (1,)), ((), ())),
                preferred_element_type=jnp.float32,
            )
            pbuf[pl.ds(c * CC, CC), :] = p.astype(jnp.bfloat16)
            if c == 0:
                pl.semaphore_wait(barrier_sem, 2)
            for s in range(K_COM // K_CMP):
                y_copy(c * (K_COM // K_CMP) + s).start()

        for k in range(K_COM):
            yc = y_copy(k)
            yc.wait_send()
            yc.wait_recv()
            out_ref[pl.ds(row0 + k * R, R), :] = (
                pbuf[pl.ds(k * R, R), :] + ybuf[pl.ds(k * R, R), :]
            )
            x_copy(k).start()

        for k in range(K_COM):
            x_copy(k).wait()

    return pl.pallas_call(
        body,
        out_shape=jax.ShapeDtypeStruct((M, N), jnp.bfloat16),
        in_specs=[
            pl.BlockSpec(memory_space=pltpu.VMEM),
            pl.BlockSpec(memory_space=pltpu.VMEM),
        ],
        out_specs=pl.BlockSpec(memory_space=pltpu.VMEM),
        scratch_shapes=[
            pltpu.VMEM((H, N), jnp.bfloat16),
            pltpu.VMEM((H, N), jnp.bfloat16),
            pltpu.SemaphoreType.DMA((K_COM,)),
            pltpu.SemaphoreType.DMA((K_COM,)),
            pltpu.SemaphoreType.DMA((K_COM,)),
            pltpu.SemaphoreType.DMA((K_COM,)),
        ],
        compiler_params=pltpu.CompilerParams(collective_id=0),
    )(dy, W)
